# baseline (speedup 1.0000x reference)
import functools

import jax
import jax.numpy as jnp
from jax import lax
from jax.experimental import pallas as pl
from jax.experimental.pallas import tpu as pltpu
from jax.experimental.pallas import tpu_sc as plsc

_N = 10000
_E = 320000
_E2 = _E + _N
_NW = 32
_BATCH = 64
_NBQ = 84
_CAPQ = 16 * _NBQ * _BATCH
_N2 = 10016
_QROWS = _N2 // 4
_AROWS = _QROWS + 8
_ZRPS = _AROWS // 16
_WRPS = _QROWS // 8
_BLK = 2504
_GRID = _N2 // _BLK


def _full_spec(shape):
    nd = len(shape)
    return pl.BlockSpec(shape, lambda i: (0,) * nd)


def _tc_first(x, Wl, bl, Wr, br):
    din, dout = Wl.shape

    def body(x_ref, wl_ref, bl_ref, wr_ref, br_ref, xl_ref, xr_ref):
        xb = x_ref[...]
        xl_ref[...] = jnp.dot(xb, wl_ref[...],
                              preferred_element_type=jnp.float32) + bl_ref[...]
        xr_ref[...] = jnp.dot(xb, wr_ref[...],
                              preferred_element_type=jnp.float32) + br_ref[...]

    return pl.pallas_call(
        body,
        grid=(_GRID,),
        in_specs=[
            pl.BlockSpec((_BLK, din), lambda i: (i, 0)),
            _full_spec((din, dout)),
            _full_spec((1, dout)),
            _full_spec((din, dout)),
            _full_spec((1, dout)),
        ],
        out_specs=[pl.BlockSpec((_BLK, dout), lambda i: (i, 0))] * 2,
        out_shape=[jax.ShapeDtypeStruct((_N2, dout), jnp.float32)] * 2,
    )(x, Wl, bl.reshape(1, -1), Wr, br.reshape(1, -1))


def _tc_mid(numer, den, bias, g, beta, Wl, bl, Wr, br):
    d = numer.shape[-1]
    din, dout = Wl.shape

    def body(na_ref, da_ref, bi_ref, g_ref, be_ref,
             wl_ref, bl_ref, wr_ref, br_ref, xl_ref, xr_ref):
        dn = da_ref[...][:, 0:1]
        h = na_ref[...] / (dn + 1e-16) + bi_ref[...]
        mu = jnp.mean(h, axis=-1, keepdims=True)
        hc = h - mu
        var = jnp.mean(hc * hc, axis=-1, keepdims=True)
        h = jnp.maximum(hc * lax.rsqrt(var + 1e-5) * g_ref[...] + be_ref[...],
                        0.0)
        xl_ref[...] = jnp.dot(h, wl_ref[...],
                              preferred_element_type=jnp.float32) + bl_ref[...]
        xr_ref[...] = jnp.dot(h, wr_ref[...],
                              preferred_element_type=jnp.float32) + br_ref[...]

    return pl.pallas_call(
        body,
        grid=(_GRID,),
        in_specs=[
            pl.BlockSpec((_BLK, d), lambda i: (i, 0)),
            pl.BlockSpec((_BLK, 16), lambda i: (i, 0)),
            _full_spec((1, d)),
            _full_spec((1, d)),
            _full_spec((1, d)),
            _full_spec((din, dout)),
            _full_spec((1, dout)),
            _full_spec((din, dout)),
            _full_spec((1, dout)),
        ],
        out_specs=[pl.BlockSpec((_BLK, dout), lambda i: (i, 0))] * 2,
        out_shape=[jax.ShapeDtypeStruct((_N2, dout), jnp.float32)] * 2,
    )(numer, den, bias.reshape(1, -1), g.reshape(1, -1), beta.reshape(1, -1),
      Wl, bl.reshape(1, -1), Wr, br.reshape(1, -1))


def _tc_post(numer, den, bias, g, beta):
    d = numer.shape[-1]

    def body(na_ref, da_ref, bi_ref, g_ref, be_ref, o_ref):
        dn = da_ref[...][:, 0:1]
        h = na_ref[...] / (dn + 1e-16) + bi_ref[...]
        mu = jnp.mean(h, axis=-1, keepdims=True)
        hc = h - mu
        var = jnp.mean(hc * hc, axis=-1, keepdims=True)
        o_ref[...] = hc * lax.rsqrt(var + 1e-5) * g_ref[...] + be_ref[...]

    return pl.pallas_call(
        body,
        grid=(_GRID,),
        in_specs=[
            pl.BlockSpec((_BLK, d), lambda i: (i, 0)),
            pl.BlockSpec((_BLK, 16), lambda i: (i, 0)),
            _full_spec((1, d)),
            _full_spec((1, d)),
            _full_spec((1, d)),
        ],
        out_specs=pl.BlockSpec((_BLK, d), lambda i: (i, 0)),
        out_shape=jax.ShapeDtypeStruct((_N2, d), jnp.float32),
    )(numer, den, bias.reshape(1, -1), g.reshape(1, -1), beta.reshape(1, -1))


@functools.lru_cache(maxsize=None)
def _sc_edge(d):
    mesh = plsc.VectorSubcoreMesh(core_axis_name="c", subcore_axis_name="s")

    @functools.partial(
        pl.kernel,
        out_type=[
            pltpu.HBM((_N2, d), jnp.float32),
            pltpu.HBM((_N2, 16), jnp.float32),
        ],
        mesh=mesh,
        compiler_params=pltpu.CompilerParams(use_tc_tiling_on_sc=False,
                                             needs_layout_passes=False),
        scratch_types=[
            pltpu.VMEM_SHARED((_AROWS, d), jnp.float32),
            pltpu.VMEM_SHARED((_AROWS, 16), jnp.float32),
        ],
    )
    def edge_kernel(xl_hbm, xr_hbm, att_hbm, idx_hbm,
                    numer_out, den_out, numer_sh, den_sh):
        pl.run_scoped(
            functools.partial(_edge_body, d, xl_hbm, xr_hbm, att_hbm,
                              idx_hbm, numer_out, den_out,
                              numer_sh, den_sh),
            pltpu.VMEM((_NBQ // 2, _BATCH), jnp.int32),
            pltpu.VMEM((3, _BATCH), jnp.int32),
            pltpu.VMEM((3, _BATCH), jnp.int32),
            pltpu.VMEM((_BATCH, d), jnp.float32),
            pltpu.VMEM((_BATCH, d), jnp.float32),
            pltpu.VMEM((_BATCH, d), jnp.float32),
            pltpu.VMEM((_BATCH, d), jnp.float32),
            pltpu.VMEM((_BATCH, 16), jnp.float32),
            pltpu.VMEM((d,), jnp.float32),
            pltpu.SemaphoreType.DMA,
            pltpu.SemaphoreType.DMA,
        )

    return edge_kernel


def _edge_body(d, xl_hbm, xr_hbm, att_hbm, idx_hbm,
               numer_out, den_out, numer_sh, den_sh,
               idx_all, idx3a, idx3b, rl0, rl1, rr0, rr1, wbuf, att_v,
               sem0, sem1):
    nch = d // 16
    hb = _NBQ // 2
    c = lax.axis_index("c")
    s = lax.axis_index("s")
    idx3 = (idx3a, idx3b)
    rl = (rl0, rl1)
    rr = (rr0, rr1)
    sems = (sem0, sem1)

    pltpu.sync_copy(att_hbm, att_v)

    zero16 = jnp.zeros((16,), jnp.float32)

    def zero_body(e, carry):
        for cc in range(nch):
            rl0[e, pl.ds(cc * 16, 16)] = zero16
        wbuf[e, :] = zero16
        return carry

    att_c = tuple(att_v[pl.ds(cc * 16, 16)] for cc in range(nch))
    lane0 = lax.iota(jnp.int32, 16) == 0

    def compute_batch(p):
        def edge_fn(e, carry):
            acc = zero16
            a_ch = []
            for cc in range(nch):
                a = rl[p][e, pl.ds(cc * 16, 16)]
                b = rr[p][e, pl.ds(cc * 16, 16)]
                a_ch.append(a)
                t = a + b
                acc = acc + jnp.maximum(t, t * 0.2) * att_c[cc]
            w16 = jnp.exp(jnp.broadcast_to(jnp.sum(acc), (16,)))
            for cc in range(nch):
                rl[p][e, pl.ds(cc * 16, 16)] = a_ch[cc] * w16
            wbuf[e, :] = jnp.where(lane0, w16, 0.0)
            return carry

        lax.fori_loop(0, _BATCH, edge_fn, 0)

    def issue(j, p, q):
        for cc in range(_BATCH // 16):
            sl = pl.ds(cc * 16, 16)
            v = idx_all[j, sl]
            raw = lax.shift_right_logical(v, 14)
            pad = raw == 16383
            idx3[p][0, sl] = v & 0x3FFF
            idx3[p][1, sl] = jnp.where(pad, 0, raw)
            idx3[p][2, sl] = jnp.where(pad, _QROWS, raw - q * _QROWS)
        pltpu.async_copy(xl_hbm.at[idx3[p].at[0]], rl[p], sems[p])
        pltpu.async_copy(xr_hbm.at[idx3[p].at[1]], rr[p], sems[p])

    def wait_gathers(p):
        pltpu.make_async_copy(
            xl_hbm.at[pl.ds(0, _BATCH)], rl[p], sems[p]).wait()
        pltpu.make_async_copy(
            xl_hbm.at[pl.ds(0, _BATCH)], rr[p], sems[p]).wait()

    for qi in range(2):
        q = c * 2 + qi

        lax.fori_loop(0, _BATCH, zero_body, 0)
        zbase = s * _ZRPS
        for off in range(0, _ZRPS, _BATCH):
            cnt = min(_BATCH, _ZRPS - off)
            pltpu.sync_copy(rl0.at[pl.ds(0, cnt)],
                            numer_sh.at[pl.ds(zbase + off, cnt)])
            pltpu.sync_copy(wbuf.at[pl.ds(0, cnt)],
                            den_sh.at[pl.ds(zbase + off, cnt)])
        plsc.subcore_barrier()

        for h in range(2):
            pltpu.sync_copy(idx_hbm.at[q, s, pl.ds(h * hb, hb)], idx_all)
            issue(0, 0, q)

            def pair_body(jj, carry):
                for p in (0, 1):
                    j = jj * 2 + p

                    @pl.when(j + 1 < hb)
                    def _():
                        issue(j + 1, 1 - p, q)

                    wait_gathers(p)
                    compute_batch(p)
                    pltpu.sync_copy(rl[p], numer_sh.at[idx3[p].at[2]],
                                    add=True)
                    pltpu.sync_copy(wbuf, den_sh.at[idx3[p].at[2]],
                                    add=True)
                return carry

            lax.fori_loop(0, hb // 2, pair_body, 0)
        plsc.subcore_barrier()

        @pl.when(s < 8)
        def _():
            wbase = s * _WRPS
            pltpu.sync_copy(
                numer_sh.at[pl.ds(wbase, _WRPS)],
                numer_out.at[pl.ds(q * _QROWS + wbase, _WRPS)])
            pltpu.sync_copy(
                den_sh.at[pl.ds(wbase, _WRPS)],
                den_out.at[pl.ds(q * _QROWS + wbase, _WRPS)])
        plsc.subcore_barrier()


def kernel(x, edge_index, Wl1, bl1, Wr1, br1, att1, bias1, g1, beta1,
           Wl2, bl2, Wr2, br2, att2, bias2, g2, beta2,
           Wl3, bl3, Wr3, br3, att3, bias3, g3, beta3):
    f32 = jnp.float32
    xp = jnp.zeros((_N2, x.shape[1]), f32).at[:_N].set(x)

    loop = jnp.arange(_N, dtype=jnp.int32)
    srcx = jnp.concatenate([edge_index[0], loop])
    dstx = jnp.concatenate([edge_index[1], loop])
    qx = dstx // _QROWS
    oh = (qx[:, None] == jnp.arange(4, dtype=jnp.int32)).astype(jnp.int32)
    rank = jnp.take_along_axis(jnp.cumsum(oh, axis=0), qx[:, None],
                               axis=1)[:, 0] - 1
    pos = jnp.where(rank < _CAPQ, qx * _CAPQ + rank, 4 * _CAPQ)
    packed = srcx | (dstx << 14)
    flat = jnp.full((4 * _CAPQ + 1,), jnp.int32(16383 << 14))
    flat = flat.at[pos].set(packed, unique_indices=True)[:-1]
    idx_q = flat.reshape(4, 16, _NBQ, _BATCH)

    xl, xr = _tc_first(xp, Wl1, bl1, Wr1, br1)
    numer, den = _sc_edge(192)(xl, xr, att1, idx_q)
    xl, xr = _tc_mid(numer, den, bias1, g1, beta1, Wl2, bl2, Wr2, br2)
    numer, den = _sc_edge(128)(xl, xr, att2, idx_q)
    xl, xr = _tc_mid(numer, den, bias2, g2, beta2, Wl3, bl3, Wr3, br3)
    numer, den = _sc_edge(64)(xl, xr, att3, idx_q)
    out = _tc_post(numer, den, bias3, g3, beta3)
    return out[:_N]


# speedup vs baseline: 1.0000x; 1.0000x over previous
import functools

import jax
import jax.numpy as jnp
from jax import lax
from jax.experimental import pallas as pl
from jax.experimental.pallas import tpu as pltpu
from jax.experimental.pallas import tpu_sc as plsc

_N = 10000
_E = 320000
_E2 = _E + _N
_NW = 32
_BATCH = 64
_NBQ = 84
_CAPQ = 16 * _NBQ * _BATCH
_N2 = 10016
_QROWS = _N2 // 4
_AROWS = _QROWS + 8
_ZRPS = _AROWS // 16
_WRPS = _QROWS // 8
_BLK = 2504
_GRID = _N2 // _BLK


def _full_spec(shape):
    nd = len(shape)
    return pl.BlockSpec(shape, lambda i: (0,) * nd)


def _tc_first(x, Wl, bl, Wr, br):
    din, dout = Wl.shape

    def body(x_ref, wl_ref, bl_ref, wr_ref, br_ref, xl_ref, xr_ref):
        xb = x_ref[...]
        xl_ref[...] = jnp.dot(xb, wl_ref[...],
                              preferred_element_type=jnp.float32) + bl_ref[...]
        xr_ref[...] = jnp.dot(xb, wr_ref[...],
                              preferred_element_type=jnp.float32) + br_ref[...]

    return pl.pallas_call(
        body,
        grid=(_GRID,),
        in_specs=[
            pl.BlockSpec((_BLK, din), lambda i: (i, 0)),
            _full_spec((din, dout)),
            _full_spec((1, dout)),
            _full_spec((din, dout)),
            _full_spec((1, dout)),
        ],
        out_specs=[pl.BlockSpec((_BLK, dout), lambda i: (i, 0))] * 2,
        out_shape=[jax.ShapeDtypeStruct((_N2, dout), jnp.float32)] * 2,
    )(x, Wl, bl.reshape(1, -1), Wr, br.reshape(1, -1))


def _tc_mid(numer, den, bias, g, beta, Wl, bl, Wr, br):
    d = numer.shape[-1]
    din, dout = Wl.shape

    def body(na_ref, da_ref, bi_ref, g_ref, be_ref,
             wl_ref, bl_ref, wr_ref, br_ref, xl_ref, xr_ref):
        dn = da_ref[...][:, 0:1]
        h = na_ref[...] / (dn + 1e-16) + bi_ref[...]
        mu = jnp.mean(h, axis=-1, keepdims=True)
        hc = h - mu
        var = jnp.mean(hc * hc, axis=-1, keepdims=True)
        h = jnp.maximum(hc * lax.rsqrt(var + 1e-5) * g_ref[...] + be_ref[...],
                        0.0)
        xl_ref[...] = jnp.dot(h, wl_ref[...],
                              preferred_element_type=jnp.float32) + bl_ref[...]
        xr_ref[...] = jnp.dot(h, wr_ref[...],
                              preferred_element_type=jnp.float32) + br_ref[...]

    return pl.pallas_call(
        body,
        grid=(_GRID,),
        in_specs=[
            pl.BlockSpec((_BLK, d), lambda i: (i, 0)),
            pl.BlockSpec((_BLK, 16), lambda i: (i, 0)),
            _full_spec((1, d)),
            _full_spec((1, d)),
            _full_spec((1, d)),
            _full_spec((din, dout)),
            _full_spec((1, dout)),
            _full_spec((din, dout)),
            _full_spec((1, dout)),
        ],
        out_specs=[pl.BlockSpec((_BLK, dout), lambda i: (i, 0))] * 2,
        out_shape=[jax.ShapeDtypeStruct((_N2, dout), jnp.float32)] * 2,
    )(numer, den, bias.reshape(1, -1), g.reshape(1, -1), beta.reshape(1, -1),
      Wl, bl.reshape(1, -1), Wr, br.reshape(1, -1))


def _tc_post(numer, den, bias, g, beta):
    d = numer.shape[-1]

    def body(na_ref, da_ref, bi_ref, g_ref, be_ref, o_ref):
        dn = da_ref[...][:, 0:1]
        h = na_ref[...] / (dn + 1e-16) + bi_ref[...]
        mu = jnp.mean(h, axis=-1, keepdims=True)
        hc = h - mu
        var = jnp.mean(hc * hc, axis=-1, keepdims=True)
        o_ref[...] = hc * lax.rsqrt(var + 1e-5) * g_ref[...] + be_ref[...]

    return pl.pallas_call(
        body,
        grid=(_GRID,),
        in_specs=[
            pl.BlockSpec((_BLK, d), lambda i: (i, 0)),
            pl.BlockSpec((_BLK, 16), lambda i: (i, 0)),
            _full_spec((1, d)),
            _full_spec((1, d)),
            _full_spec((1, d)),
        ],
        out_specs=pl.BlockSpec((_BLK, d), lambda i: (i, 0)),
        out_shape=jax.ShapeDtypeStruct((_N2, d), jnp.float32),
    )(numer, den, bias.reshape(1, -1), g.reshape(1, -1), beta.reshape(1, -1))


@functools.lru_cache(maxsize=None)
def _sc_edge(d):
    mesh = plsc.VectorSubcoreMesh(core_axis_name="c", subcore_axis_name="s")

    @functools.partial(
        pl.kernel,
        out_type=[
            pltpu.HBM((_N2, d), jnp.float32),
            pltpu.HBM((_N2, 16), jnp.float32),
        ],
        mesh=mesh,
        compiler_params=pltpu.CompilerParams(use_tc_tiling_on_sc=False,
                                             needs_layout_passes=False),
        scratch_types=[
            pltpu.VMEM_SHARED((_AROWS, d), jnp.float32),
            pltpu.VMEM_SHARED((_AROWS, 16), jnp.float32),
        ],
    )
    def edge_kernel(xl_hbm, xr_hbm, att_hbm, idx_hbm,
                    numer_out, den_out, numer_sh, den_sh):
        pl.run_scoped(
            functools.partial(_edge_body, d, xl_hbm, xr_hbm, att_hbm,
                              idx_hbm, numer_out, den_out,
                              numer_sh, den_sh),
            pltpu.VMEM((_NBQ // 2, _BATCH), jnp.int32),
            pltpu.VMEM((3, _BATCH), jnp.int32),
            pltpu.VMEM((3, _BATCH), jnp.int32),
            pltpu.VMEM((_BATCH, d), jnp.float32),
            pltpu.VMEM((_BATCH, d), jnp.float32),
            pltpu.VMEM((_BATCH, d), jnp.float32),
            pltpu.VMEM((_BATCH, d), jnp.float32),
            pltpu.VMEM((_BATCH, 16), jnp.float32),
            pltpu.VMEM((d,), jnp.float32),
            pltpu.SemaphoreType.DMA,
            pltpu.SemaphoreType.DMA,
        )

    return edge_kernel


def _edge_body(d, xl_hbm, xr_hbm, att_hbm, idx_hbm,
               numer_out, den_out, numer_sh, den_sh,
               idx_all, idx3a, idx3b, rl0, rl1, rr0, rr1, wbuf, att_v,
               sem0, sem1):
    nch = d // 16
    hb = _NBQ // 2
    c = lax.axis_index("c")
    s = lax.axis_index("s")
    idx3 = (idx3a, idx3b)
    rl = (rl0, rl1)
    rr = (rr0, rr1)
    sems = (sem0, sem1)

    pltpu.sync_copy(att_hbm, att_v)

    zero16 = jnp.zeros((16,), jnp.float32)

    def zero_body(e, carry):
        for cc in range(nch):
            rl0[e, pl.ds(cc * 16, 16)] = zero16
        wbuf[e, :] = zero16
        return carry

    att_c = tuple(att_v[pl.ds(cc * 16, 16)] for cc in range(nch))
    lane0 = lax.iota(jnp.int32, 16) == 0

    def compute_batch(p):
        def edge_fn(e, carry):
            acc = zero16
            a_ch = []
            for cc in range(nch):
                a = rl[p][e, pl.ds(cc * 16, 16)]
                b = rr[p][e, pl.ds(cc * 16, 16)]
                a_ch.append(a)
                t = a + b
                acc = acc + jnp.maximum(t, t * 0.2) * att_c[cc]
            w16 = jnp.exp(jnp.broadcast_to(jnp.sum(acc), (16,)))
            for cc in range(nch):
                rl[p][e, pl.ds(cc * 16, 16)] = a_ch[cc] * w16
            wbuf[e, :] = jnp.where(lane0, w16, 0.0)
            return carry

        lax.fori_loop(0, _BATCH, edge_fn, 0)

    def issue(j, p, q):
        for cc in range(_BATCH // 16):
            sl = pl.ds(cc * 16, 16)
            v = idx_all[j, sl]
            raw = lax.shift_right_logical(v, 14)
            pad = raw == 16383
            idx3[p][0, sl] = v & 0x3FFF
            idx3[p][1, sl] = jnp.where(pad, 0, raw)
            idx3[p][2, sl] = jnp.where(pad, _QROWS, raw - q * _QROWS)
        pltpu.async_copy(xl_hbm.at[idx3[p].at[0]], rl[p], sems[p])
        pltpu.async_copy(xr_hbm.at[idx3[p].at[1]], rr[p], sems[p])

    def wait_gathers(p):
        pltpu.make_async_copy(
            xl_hbm.at[pl.ds(0, _BATCH)], rl[p], sems[p]).wait()
        pltpu.make_async_copy(
            xl_hbm.at[pl.ds(0, _BATCH)], rr[p], sems[p]).wait()

    for qi in range(2):
        q = c * 2 + qi

        lax.fori_loop(0, _BATCH, zero_body, 0)
        zbase = s * _ZRPS
        for off in range(0, _ZRPS, _BATCH):
            cnt = min(_BATCH, _ZRPS - off)
            pltpu.sync_copy(rl0.at[pl.ds(0, cnt)],
                            numer_sh.at[pl.ds(zbase + off, cnt)])
            pltpu.sync_copy(wbuf.at[pl.ds(0, cnt)],
                            den_sh.at[pl.ds(zbase + off, cnt)])
        plsc.subcore_barrier()

        for h in range(2):
            pltpu.sync_copy(idx_hbm.at[q, s, pl.ds(h * hb, hb)], idx_all)
            issue(0, 0, q)

            def pair_body(jj, carry):
                for p in (0, 1):
                    j = jj * 2 + p

                    @pl.when(j + 1 < hb)
                    def _():
                        issue(j + 1, 1 - p, q)

                    wait_gathers(p)
                    compute_batch(p)
                    pltpu.sync_copy(rl[p], numer_sh.at[idx3[p].at[2]],
                                    add=True)
                    pltpu.sync_copy(wbuf, den_sh.at[idx3[p].at[2]],
                                    add=True)
                return carry

            lax.fori_loop(0, hb // 2, pair_body, 0)
        plsc.subcore_barrier()

        @pl.when(s < 8)
        def _():
            wbase = s * _WRPS
            pltpu.sync_copy(
                numer_sh.at[pl.ds(wbase, _WRPS)],
                numer_out.at[pl.ds(q * _QROWS + wbase, _WRPS)])
            pltpu.sync_copy(
                den_sh.at[pl.ds(wbase, _WRPS)],
                den_out.at[pl.ds(q * _QROWS + wbase, _WRPS)])
        plsc.subcore_barrier()


def kernel(x, edge_index, Wl1, bl1, Wr1, br1, att1, bias1, g1, beta1,
           Wl2, bl2, Wr2, br2, att2, bias2, g2, beta2,
           Wl3, bl3, Wr3, br3, att3, bias3, g3, beta3):
    f32 = jnp.float32
    xp = jnp.zeros((_N2, x.shape[1]), f32).at[:_N].set(x)

    loop = jnp.arange(_N, dtype=jnp.int32)
    srcx = jnp.concatenate([edge_index[0], loop])
    dstx = jnp.concatenate([edge_index[1], loop])
    qx = dstx // _QROWS
    oh = (qx[None, :] == jnp.arange(4, dtype=jnp.int32)[:, None]
          ).astype(jnp.int32)
    rank = jnp.take_along_axis(jnp.cumsum(oh, axis=1), qx[None, :],
                               axis=0)[0] - 1
    pos = jnp.where(rank < _CAPQ, qx * _CAPQ + rank, 4 * _CAPQ)
    packed = srcx | (dstx << 14)
    flat = jnp.full((4 * _CAPQ + 1,), jnp.int32(16383 << 14))
    flat = flat.at[pos].set(packed, unique_indices=True)[:-1]
    idx_q = flat.reshape(4, 16, _NBQ, _BATCH)

    xl, xr = _tc_first(xp, Wl1, bl1, Wr1, br1)
    numer, den = _sc_edge(192)(xl, xr, att1, idx_q)
    xl, xr = _tc_mid(numer, den, bias1, g1, beta1, Wl2, bl2, Wr2, br2)
    numer, den = _sc_edge(128)(xl, xr, att2, idx_q)
    xl, xr = _tc_mid(numer, den, bias2, g2, beta2, Wl3, bl3, Wr3, br3)
    numer, den = _sc_edge(64)(xl, xr, att3, idx_q)
    out = _tc_post(numer, den, bias3, g3, beta3)
    return out[:_N]
